# Initial kernel scaffold; baseline (speedup 1.0000x reference)
import jax
import jax.numpy as jnp
from jax.experimental import pallas as pl


def kernel(x, edge_index, e, i, W1, b1, gamma1, beta1, W2, b2, gamma2, beta2, Wd, bd):
    raise NotImplementedError("write your pallas kernel here")


# speedup vs baseline: 13.6833x; 13.6833x over previous
import functools

import jax
import jax.numpy as jnp
from jax import lax
from jax.experimental import pallas as pl
from jax.experimental.pallas import tpu as pltpu
from jax.experimental.pallas import tpu_sc as plsc

N = 10000
F = 256
C = 16
G = 8
L = 12

NC = 2
NS = 16
NW = NC * NS
CHUNK = 128
NPAD = N + 16

ROWS_T = N // NS
ROWS_TP = NPAD // NS


def _mm_body(x_ref, w_ref, b_ref, o_ref):
    o_ref[...] = (
        jnp.dot(x_ref[...], w_ref[...], preferred_element_type=jnp.float32)
        + b_ref[...]
    )


def _dense1(x, W1, b1):
    return pl.pallas_call(
        _mm_body,
        grid=(10,),
        in_specs=[
            pl.BlockSpec((N // 10, F), lambda m: (m, 0)),
            pl.BlockSpec((F, C), lambda m: (0, 0)),
            pl.BlockSpec((1, C), lambda m: (0, 0)),
        ],
        out_specs=pl.BlockSpec((N // 10, C), lambda m: (m, 0)),
        out_shape=jax.ShapeDtypeStruct((N, C), jnp.float32),
    )(x, W1, b1.reshape(1, C))


def _sc_agg_body(nchunk, h_hbm, zeros_hbm, src_hbm, dst_hbm, out_hbm,
                 src_v, dst_v, rows_v, h_sp, acc_sp):
    c = lax.axis_index("c")
    s = lax.axis_index("s")
    wid = c * NS + s

    @pl.when(s == 0)
    def _():
        pltpu.sync_copy(h_hbm, h_sp)
        pltpu.sync_copy(zeros_hbm, acc_sp)

    pltpu.sync_copy(src_hbm.at[wid], src_v)
    pltpu.sync_copy(dst_hbm.at[wid], dst_v)
    plsc.subcore_barrier()

    def body(j, carry):
        pltpu.sync_copy(h_sp.at[src_v.at[j]], rows_v)
        pltpu.sync_copy(rows_v, acc_sp.at[dst_v.at[j]], add=True)
        return carry

    lax.fori_loop(0, nchunk, body, 0)
    plsc.subcore_barrier()

    @pl.when(s == 0)
    def _():
        pltpu.sync_copy(acc_sp.at[pl.ds(0, N)], out_hbm.at[c])


def _sc_agg(h, zeros, src_p, dst_p, nchunk):
    mesh = plsc.VectorSubcoreMesh(core_axis_name="c", subcore_axis_name="s")
    fn = pl.kernel(
        functools.partial(_sc_agg_body, nchunk),
        mesh=mesh,
        out_type=jax.ShapeDtypeStruct((NC, N, C), jnp.float32),
        scratch_types=[
            pltpu.VMEM((nchunk, CHUNK), jnp.int32),
            pltpu.VMEM((nchunk, CHUNK), jnp.int32),
            pltpu.VMEM((CHUNK, C), jnp.float32),
            pltpu.VMEM_SHARED((N, C), jnp.float32),
            pltpu.VMEM_SHARED((NPAD, C), jnp.float32),
        ],
        compiler_params=pltpu.CompilerParams(use_tc_tiling_on_sc=False),
    )
    return fn(h, zeros, src_p, dst_p)


def _bn_mm_body(p_ref, g_ref, be_ref, w_ref, b_ref, o_ref):
    agg = p_ref[0] + p_ref[1]
    mean = jnp.mean(agg, axis=0, keepdims=True)
    var = jnp.mean((agg - mean) ** 2, axis=0, keepdims=True)
    hn = (agg - mean) / jnp.sqrt(var + 1e-3) * g_ref[...] + be_ref[...]
    h = jnp.maximum(hn, 0.0)
    o_ref[...] = (
        jnp.dot(h, w_ref[...], preferred_element_type=jnp.float32) + b_ref[...]
    )


def _fuse_bn_mm(partial, gamma, beta, W2, b2):
    return pl.pallas_call(
        _bn_mm_body,
        out_shape=jax.ShapeDtypeStruct((N, C), jnp.float32),
    )(partial, gamma.reshape(1, C), beta.reshape(1, C), W2, b2.reshape(1, C))


def _bn_pool_body(p_ref, i_ref, g_ref, be_ref, w_ref, b_ref, o_ref):
    agg = p_ref[0] + p_ref[1]
    mean = jnp.mean(agg, axis=0, keepdims=True)
    var = jnp.mean((agg - mean) ** 2, axis=0, keepdims=True)
    hn = (agg - mean) / jnp.sqrt(var + 1e-3) * g_ref[...] + be_ref[...]
    h = jnp.maximum(hn, 0.0)
    gid = i_ref[...]
    pooled = jnp.concatenate(
        [
            jnp.sum(jnp.where(gid == g, h, 0.0), axis=0, keepdims=True)
            for g in range(G)
        ],
        axis=0,
    )
    z = jnp.dot(pooled, w_ref[...], preferred_element_type=jnp.float32) + b_ref[...]
    o_ref[...] = jax.nn.sigmoid(z)


def _fuse_bn_pool(partial, i2d, gamma, beta, Wd, bd):
    return pl.pallas_call(
        _bn_pool_body,
        out_shape=jax.ShapeDtypeStruct((G, L), jnp.float32),
    )(partial, i2d, gamma.reshape(1, C), beta.reshape(1, C), Wd,
      bd.reshape(1, L))


def kernel(x, edge_index, e, i, W1, b1, gamma1, beta1, W2, b2, gamma2, beta2,
           Wd, bd):
    del e
    E = edge_index.shape[1]
    nchunk = -(-E // (NW * CHUNK))
    e_pad = NW * nchunk * CHUNK
    src = edge_index[0].astype(jnp.int32)
    dst = edge_index[1].astype(jnp.int32)
    src_p = jnp.concatenate(
        [src, jnp.zeros((e_pad - E,), jnp.int32)]).reshape(NW, nchunk, CHUNK)
    dst_p = jnp.concatenate(
        [dst, jnp.full((e_pad - E,), N, jnp.int32)]).reshape(NW, nchunk, CHUNK)
    zeros = jnp.zeros((NPAD, C), jnp.float32)

    h1 = _dense1(x, W1, b1)
    p1 = _sc_agg(h1, zeros, src_p, dst_p, nchunk)
    h2 = _fuse_bn_mm(p1, gamma1, beta1, W2, b2)
    p2 = _sc_agg(h2, zeros, src_p, dst_p, nchunk)
    return _fuse_bn_pool(p2, i.astype(jnp.int32).reshape(N, 1), gamma2, beta2,
                         Wd, bd)
